# baseline (speedup 1.0000x reference)
import jax
import jax.numpy as jnp
from jax.experimental import pallas as pl
from jax.experimental.pallas import tpu as pltpu


_CIN1, _COUT1, _K1 = 3, 3, 5
_COUT2, _K2 = 5, 3
_H, _W = 32, 32
_OH1, _OW1 = _H - _K1 + 1, _W - _K1 + 1
_PH1, _PW1 = _OH1 // 2, _OW1 // 2
_OH2, _OW2 = _PH1 - _K2 + 1, _PW1 - _K2 + 1
_PH2, _PW2 = _OH2 // 2, _OW2 // 2
_NHID, _NOUT = 100, 10

_RB1 = 4
_NB1 = _OH1 // _RB1
_XR1 = _RB1 + _K1 - 1
_CHUNK1 = _XR1 * _W
_G1 = 64
_M1 = 2 * 2 * 2 * _G1
_F1L = _PH1 * _G1
_G2 = 256
_M2 = 2 * 2 * _G2


def _prep_operands(w1, b1, w2, b2, wf1, bf1, wf2, bf2):
    f32 = jnp.float32
    w1r = w1.astype(f32).reshape(_COUT1, _CIN1, _K1, _K1)
    w2r = w2.astype(f32).reshape(_COUT2, _CIN1, _K2, _K2)

    a1 = (jnp.arange(_XR1)[None, None, :, None]
          - (2 * jnp.arange(2)[None, :, None, None]
             + jnp.arange(2)[:, None, None, None])
          == jnp.arange(_K1)[None, None, None, :]).astype(f32)
    c1 = (jnp.arange(_W)[None, None, :, None]
          - (2 * jnp.arange(_PW1)[None, :, None, None]
             + jnp.arange(2)[:, None, None, None])
          == jnp.arange(_K1)[None, None, None, :]).astype(f32)
    band1 = jnp.einsum("oikl,pjhk,xqwl->xpjoqihw", w1r, a1, c1)
    band1 = band1.reshape(2 * 2 * 2, _COUT1 * _PW1, _CIN1, _XR1 * _W)
    band1 = jnp.pad(band1, ((0, 0), (0, _G1 - _COUT1 * _PW1), (0, 0), (0, 0)))
    band1t = band1.transpose(2, 3, 0, 1).reshape(_CIN1 * _CHUNK1, _M1)

    b1vec = jnp.broadcast_to(b1.astype(f32).reshape(1, _COUT1, 1),
                             (2, _COUT1, _PW1)).reshape(2, _COUT1 * _PW1)
    b1vec = jnp.pad(b1vec, ((0, 0), (0, _G1 - _COUT1 * _PW1))).reshape(1, 2 * _G1)

    a2 = (jnp.arange(_PH1)[None, None, :, None]
          - (2 * jnp.arange(_PH2)[None, :, None, None]
             + jnp.arange(2)[:, None, None, None])
          == jnp.arange(_K2)[None, None, None, :]).astype(f32)
    c2 = (jnp.arange(_PW1)[None, None, :, None]
          - (2 * jnp.arange(_PW2)[None, :, None, None]
             + jnp.arange(2)[:, None, None, None])
          == jnp.arange(_K2)[None, None, None, :]).astype(f32)
    band2 = jnp.einsum("oikl,pjhk,xsql->xpojshiq", w2r, a2, c2)
    band2 = band2.reshape(2 * 2, _COUT2 * _PH2 * _PW2, _PH1, _CIN1 * _PW1)
    band2 = jnp.pad(band2, ((0, 0), (0, _G2 - _COUT2 * _PH2 * _PW2),
                            (0, 0), (0, _G1 - _CIN1 * _PW1)))
    band2t = band2.transpose(2, 3, 0, 1).reshape(_F1L, _M2)

    b2vec = jnp.broadcast_to(b2.astype(f32).reshape(_COUT2, 1),
                             (_COUT2, _PH2 * _PW2)).reshape(1, -1)
    b2vec = jnp.pad(b2vec, ((0, 0), (0, _G2 - _COUT2 * _PH2 * _PW2)))

    wf1t = jnp.pad(wf1.astype(f32).T,
                   ((0, _G2 - _COUT2 * _PH2 * _PW2), (0, 0)))
    wf2t = wf2.astype(f32).T
    bf1v = bf1.astype(f32).reshape(1, _NHID)
    bf2v = bf2.astype(f32).reshape(1, _NOUT)
    return band1t, b1vec, band2t, b2vec, wf1t, bf1v, wf2t, bf2v


def _net_kernel(x_ref, wb1_ref, b1_ref, wb2_ref, b2_ref,
                wf1_ref, bf1_ref, wf2_ref, bf2_ref,
                o_ref, f1_ref):
    for blk in range(_NB1):
        z = None
        for ci in range(_CIN1):
            lo = ci * _H * _W + blk * _RB1 * _W
            xs = x_ref[:, lo:lo + _CHUNK1]
            wb = wb1_ref[ci * _CHUNK1:(ci + 1) * _CHUNK1, :]
            t = jnp.dot(xs, wb, preferred_element_type=jnp.float32)
            z = t if z is None else z + t
        pooled = jnp.maximum(jnp.maximum(z[:, 0:128], z[:, 128:256]),
                             jnp.maximum(z[:, 256:384], z[:, 384:512]))
        act = jnp.maximum(pooled + b1_ref[...], 0.0)
        f1_ref[:, 2 * blk * _G1:2 * blk * _G1 + 2 * _G1] = act

    z2 = jnp.dot(f1_ref[...], wb2_ref[...],
                 preferred_element_type=jnp.float32)
    pooled2 = jnp.maximum(jnp.maximum(z2[:, 0:256], z2[:, 256:512]),
                          jnp.maximum(z2[:, 512:768], z2[:, 768:1024]))
    feats = jnp.maximum(pooled2 + b2_ref[...], 0.0)

    h = jnp.dot(feats, wf1_ref[...], preferred_element_type=jnp.float32)
    h = jnp.maximum(h + bf1_ref[...], 0.0)
    o = jnp.dot(h, wf2_ref[...], preferred_element_type=jnp.float32)
    o_ref[...] = o + bf2_ref[...]


def kernel(x, w1, b1, w2, b2, wf1, bf1, wf2, bf2):
    ops = _prep_operands(w1, b1, w2, b2, wf1, bf1, wf2, bf2)
    band1t, b1vec, band2t, b2vec, wf1t, bf1v, wf2t, bf2v = ops

    n = x.shape[0]
    tile_n = n if n <= 256 else 256
    n_pad = ((n + tile_n - 1) // tile_n) * tile_n

    x_flat = x.astype(jnp.float32).reshape(n, _CIN1 * _H * _W)
    if n_pad != n:
        x_flat = jnp.pad(x_flat, ((0, n_pad - n), (0, 0)))

    out = pl.pallas_call(
        _net_kernel,
        out_shape=jax.ShapeDtypeStruct((n_pad, _NOUT), jnp.float32),
        grid=(n_pad // tile_n,),
        in_specs=[
            pl.BlockSpec((tile_n, _CIN1 * _H * _W), lambda i: (i, 0)),
            pl.BlockSpec((_CIN1 * _CHUNK1, _M1), lambda i: (0, 0)),
            pl.BlockSpec((1, 2 * _G1), lambda i: (0, 0)),
            pl.BlockSpec((_F1L, _M2), lambda i: (0, 0)),
            pl.BlockSpec((1, _G2), lambda i: (0, 0)),
            pl.BlockSpec((_G2, _NHID), lambda i: (0, 0)),
            pl.BlockSpec((1, _NHID), lambda i: (0, 0)),
            pl.BlockSpec((_NHID, _NOUT), lambda i: (0, 0)),
            pl.BlockSpec((1, _NOUT), lambda i: (0, 0)),
        ],
        out_specs=pl.BlockSpec((tile_n, _NOUT), lambda i: (i, 0)),
        scratch_shapes=[
            pltpu.VMEM((tile_n, _F1L), jnp.float32),
        ],
        compiler_params=pltpu.CompilerParams(
            dimension_semantics=("parallel",),
            vmem_limit_bytes=48 * 1024 * 1024,
        ),
    )(x_flat, band1t, b1vec, band2t, b2vec, wf1t, bf1v, wf2t, bf2v)

    return out[:n]
